# baseline (speedup 1.0000x reference)
import functools

import jax
import jax.numpy as jnp
from jax import lax
from jax.experimental import pallas as pl
from jax.experimental.pallas import tpu as pltpu
from jax.experimental.pallas import tpu_sc as plsc

_V = 1000000
_VP = _V // 2
_D = 64
_B = 4096
_S = 200
_NC = 2
_NS = 16
_NW = _NC * _NS

_VMAIN = 999936
_CHUNK_V = 384
_N_CHUNKS = _VMAIN // _CHUNK_V
_SG = _S // 8

_mesh = plsc.VectorSubcoreMesh(core_axis_name="c", subcore_axis_name="s")
_params = pltpu.CompilerParams(use_tc_tiling_on_sc=True, needs_layout_passes=False)


@functools.partial(
    pl.kernel,
    mesh=_mesh,
    out_type=jax.ShapeDtypeStruct((_VP, 128), jnp.float32),
    scratch_types=[
        pltpu.VMEM((_D, _CHUNK_V), jnp.float32),
        pltpu.VMEM((_CHUNK_V // 2, 128), jnp.float32),
        pltpu.SemaphoreType.DMA,
    ],
    compiler_params=_params,
)
def _repack(wt_hbm, tail_hbm, pk_hbm, in_v, out_v, sem):
    wid = lax.axis_index("s") * _NC + lax.axis_index("c")

    @pl.when(wid == 0)
    def _tail():
        pltpu.async_copy(tail_hbm, out_v.at[pl.ds(0, 32)], sem).wait()
        pltpu.async_copy(
            out_v.at[pl.ds(0, 32)], pk_hbm.at[pl.ds(_VP - 32, 32)], sem
        ).wait()

    iota16 = lax.iota(jnp.int32, 16)
    rowpat = lax.shift_right_logical(iota16, 1)
    colpat = lax.mul(lax.rem(iota16, 2), jnp.int32(64))

    def body(i, carry):
        c = i * _NW + wid

        @pl.when(c < _N_CHUNKS)
        def _work():
            v0 = pl.multiple_of(c * _CHUNK_V, 128)
            pltpu.async_copy(wt_hbm.at[:, pl.ds(v0, _CHUNK_V)], in_v, sem).wait()

            def dloop(d, carry2):
                def gloop(g, carry3):
                    vec = in_v[d, pl.ds(pl.multiple_of(g * 16, 16), 16)]
                    rows = rowpat + g * 8
                    cols = colpat + d
                    plsc.store_scatter(out_v, [rows, cols], vec)
                    return carry3

                lax.fori_loop(0, _CHUNK_V // 16, gloop, 0)
                return carry2

            lax.fori_loop(0, _D, dloop, 0)
            p0 = pl.multiple_of(c * (_CHUNK_V // 2), 8)
            pltpu.async_copy(out_v, pk_hbm.at[pl.ds(p0, _CHUNK_V // 2)], sem).wait()

        return carry

    lax.fori_loop(0, (_N_CHUNKS + _NW - 1) // _NW, body, 0)


@functools.partial(
    pl.kernel,
    mesh=_mesh,
    out_type=jax.ShapeDtypeStruct((_S, _D, _B), jnp.float32),
    scratch_types=[
        pltpu.VMEM((8, 128), jnp.int32),
        pltpu.VMEM((8, 128), jnp.int32),
        pltpu.VMEM((2, 128, 128), jnp.float32),
        pltpu.VMEM((2, _D, 128), jnp.float32),
        pltpu.SemaphoreType.DMA,
        pltpu.SemaphoreType.DMA,
        pltpu.SemaphoreType.DMA,
    ],
    compiler_params=_params,
)
def _gather_t(pidx_hbm, hcol_hbm, pk_hbm, ot_hbm, idx_v, h_v, pair_v, ot_v,
              sem_i, sem_g, sem_w):
    wid = lax.axis_index("s") * _NC + lax.axis_index("c")
    b0 = pl.multiple_of(wid * 128, 128)
    iota16 = lax.iota(jnp.int32, 16)

    def sgroup(sg, carry):
        s0 = sg * 8
        pltpu.async_copy(pidx_hbm.at[pl.ds(s0, 8), pl.ds(b0, 128)], idx_v, sem_i)
        pltpu.async_copy(hcol_hbm.at[pl.ds(s0, 8), pl.ds(b0, 128)], h_v, sem_i)
        pltpu.make_async_copy(pidx_hbm.at[pl.ds(s0, 8), pl.ds(b0, 128)], idx_v, sem_i).wait()
        pltpu.make_async_copy(hcol_hbm.at[pl.ds(s0, 8), pl.ds(b0, 128)], h_v, sem_i).wait()

        gathers = []
        writes = []
        for sp in range(8):
            pb = sp % 2
            ob = sp % 2
            if sp == 0:
                gathers.append(
                    pltpu.async_copy(pk_hbm.at[idx_v.at[0]], pair_v.at[0], sem_g)
                )
                gathers.append(
                    pltpu.async_copy(pk_hbm.at[idx_v.at[1]], pair_v.at[1], sem_g)
                )
            gathers[sp].wait()

            if sp >= 2:
                writes[sp - 2].wait()

            hcols = [h_v[sp, pl.ds(g * 16, 16)] for g in range(8)]

            def dloop(d, carry2):
                for g in range(8):
                    vec = plsc.load_gather(
                        pair_v.at[pb], [g * 16 + iota16, hcols[g] + d]
                    )
                    ot_v[ob, d, pl.ds(g * 16, 16)] = vec
                return carry2

            lax.fori_loop(0, _D, dloop, 0)

            writes.append(
                pltpu.async_copy(
                    ot_v.at[ob], ot_hbm.at[s0 + sp, :, pl.ds(b0, 128)], sem_w
                )
            )
            if sp < 6:
                gathers.append(
                    pltpu.async_copy(
                        pk_hbm.at[idx_v.at[sp + 2]], pair_v.at[(sp + 2) % 2], sem_g
                    )
                )
        writes[6].wait()
        writes[7].wait()
        return carry

    lax.fori_loop(0, _SG, sgroup, 0)


def kernel(sentence, elmo_tensor, word_embed):
    del elmo_tensor
    wt = word_embed.T
    tail = word_embed[_VMAIN:, :].reshape(32, 128)
    pk = _repack(wt, tail)
    st = sentence.T
    pidx = lax.shift_right_logical(st, 1)
    hcol = lax.shift_left(lax.bitwise_and(st, 1), 6)
    ot = _gather_t(pidx, hcol, pk)
    return ot.transpose(2, 0, 1)
